# baseline (speedup 1.0000x reference)
import jax
import jax.numpy as jnp
from jax.experimental import pallas as pl
from jax.experimental.pallas import tpu as pltpu

_B_TILE = 512
_NC = 1000
_NC_PAD = 1024


def _fused_kernel(lang_ref, sens_ref, lw1_ref, lb1_ref, lw2_ref, lb2_ref,
                  sw1_ref, sb1_ref, sw2_ref, sb2_ref, ceT_ref,
                  gw1l_ref, gw1s_ref, gb1_ref, gw2_ref, gb2_ref,
                  idx_ref, score_ref):
    f32 = jnp.float32
    mm = lambda a, b: jnp.dot(a, b, preferred_element_type=f32)
    def rne_bf16(x):
        u = jax.lax.bitcast_convert_type(x, jnp.uint32)
        r = (u + jnp.uint32(0x7FFF) + ((u >> 16) & jnp.uint32(1)))
        r = r & jnp.uint32(0xFFFF0000)
        return jax.lax.bitcast_convert_type(r, f32)

    lh = jnp.maximum(mm(lang_ref[:], lw1_ref[:]) + lb1_ref[:], 0.0)
    lc = mm(rne_bf16(lh), lw2_ref[:]) + lb2_ref[:]
    sh = jnp.maximum(mm(sens_ref[:], sw1_ref[:]) + sb1_ref[:], 0.0)
    sc = mm(rne_bf16(sh), sw2_ref[:]) + sb2_ref[:]
    lc_r = rne_bf16(lc)
    sc_r = rne_bf16(sc)
    g = jnp.maximum(
        mm(lc_r, gw1l_ref[:]) + mm(sc_r, gw1s_ref[:]) + gb1_ref[:], 0.0)
    pre = jnp.sum(g * gw2_ref[:], axis=-1, keepdims=True) + gb2_ref[:]
    score_ref[:] = jax.nn.sigmoid(pre)

    ceT = ceT_ref[:]
    cnorm = jnp.sum(ceT * ceT, axis=0, keepdims=True)
    col1 = jax.lax.broadcasted_iota(jnp.int32, (1, _NC_PAD), 1)
    cnorm = jnp.where(col1 < _NC, cnorm, jnp.float32(3e38))
    lcnorm = jnp.sum(lc * lc, axis=-1, keepdims=True)
    d2 = lcnorm + cnorm - 2.0 * mm(lc_r, ceT)
    metric = jnp.sqrt(jnp.maximum(d2, 0.0))
    mins = jnp.min(metric, axis=-1, keepdims=True)
    cols = jax.lax.broadcasted_iota(jnp.int32, metric.shape, 1)
    idx_ref[:] = jnp.min(jnp.where(metric == mins, cols, _NC_PAD),
                         axis=-1, keepdims=True)


def kernel(language_input, sensorimotor_input, l_w1, l_b1, l_w2, l_b2,
           s_w1, s_b1, s_w2, s_b2, concept_embeddings, g_w1, g_b1, g_w2,
           g_b2):
    B, LD = language_input.shape
    SD = sensorimotor_input.shape[1]
    CD = l_w2.shape[1]
    ceT = jnp.pad(concept_embeddings, ((0, _NC_PAD - _NC), (0, 0))).T
    args = (
        language_input, sensorimotor_input,
        l_w1, l_b1.reshape(1, -1), l_w2, l_b2.reshape(1, -1),
        s_w1, s_b1.reshape(1, -1), s_w2, s_b2.reshape(1, -1),
        ceT,
        g_w1[:CD], g_w1[CD:], g_b1.reshape(1, -1),
        g_w2.reshape(1, -1), g_b2.reshape(1, 1),
    )
    full = lambda a: pl.BlockSpec(a.shape, lambda i: (0,) * a.ndim)
    in_specs = [
        pl.BlockSpec((_B_TILE, LD), lambda i: (i, 0)),
        pl.BlockSpec((_B_TILE, SD), lambda i: (i, 0)),
    ] + [full(a) for a in args[2:]]
    idx2d, score = pl.pallas_call(
        _fused_kernel,
        grid=(B // _B_TILE,),
        in_specs=in_specs,
        out_specs=[
            pl.BlockSpec((_B_TILE, 1), lambda i: (i, 0)),
            pl.BlockSpec((_B_TILE, 1), lambda i: (i, 0)),
        ],
        out_shape=[
            jax.ShapeDtypeStruct((B, 1), jnp.int32),
            jax.ShapeDtypeStruct((B, 1), jnp.float32),
        ],
        compiler_params=pltpu.CompilerParams(
            dimension_semantics=("parallel",)),
    )(*args)
    return idx2d.reshape(B), score
